# Initial kernel scaffold; baseline (speedup 1.0000x reference)
import jax
import jax.numpy as jnp
from jax.experimental import pallas as pl


def kernel(flat_ids, cu_seqlens):
    raise NotImplementedError("write your pallas kernel here")


# speedup vs baseline: 9.9168x; 9.9168x over previous
import functools

import jax
import jax.numpy as jnp
from jax import lax
from jax.experimental import pallas as pl
from jax.experimental.pallas import tpu as pltpu
from jax.experimental.pallas import tpu_sc as plsc

B = 16
TOTAL = 32768
CLS_ID = 101
SEP_ID = 102
PADLEN = 4098
MAXTOK = PADLEN - 2

HALF = 2064
W = 2 * HALF
NVEC = HALF // 16
STAGE = HALF + 8
PAD = 8
BUF = STAGE + PAD + 16

_mesh = plsc.VectorSubcoreMesh(core_axis_name="c", subcore_axis_name="s")


@functools.partial(
    pl.kernel,
    out_type=jax.ShapeDtypeStruct((B, 2, HALF), jnp.int32),
    mesh=_mesh,
    scratch_types=[
        pltpu.VMEM((48,), jnp.int32),
        pltpu.VMEM((BUF,), jnp.int32),
        pltpu.VMEM((HALF,), jnp.int32),
    ],
)
def _pad_rows(params_hbm, flat_hbm, out_hbm, params_v, stage_v, row_v):
    h = lax.axis_index("c")
    r = lax.axis_index("s")

    pltpu.sync_copy(params_hbm, params_v)

    lane = lax.iota(jnp.int32, 16)
    start = params_v[pl.ds(r, 16)][0]
    clen = params_v[pl.ds(r + 16, 16)][0]

    c0 = h * HALF
    src_lo = start + c0 - 1
    abase = jnp.clip((jnp.maximum(src_lo, 0) // 8) * 8, 0, TOTAL - STAGE)
    abase = pl.multiple_of(abase, 8)
    pltpu.sync_copy(flat_hbm.at[pl.ds(abase, STAGE)], stage_v.at[pl.ds(PAD, STAGE)])

    shift = src_lo - abase + PAD
    sep_col = clen + 1

    def body(j, carry):
        l = j * 16 + lane
        col = c0 + l
        base = jnp.clip(shift + j * 16, 0, BUF - 16)
        tok = stage_v[pl.ds(base, 16)]
        val = jnp.where(col == 0, jnp.int32(CLS_ID),
              jnp.where(col == sep_col, jnp.int32(SEP_ID),
              jnp.where(col <= clen, tok, jnp.int32(0))))
        row_v[pl.ds(j * 16, 16)] = val
        return carry

    lax.fori_loop(0, NVEC, body, 0)
    pltpu.sync_copy(row_v, out_hbm.at[r, h])


def kernel(flat_ids, cu_seqlens):
    starts = cu_seqlens[:B]
    clens = jnp.minimum(cu_seqlens[1:] - cu_seqlens[:-1], MAXTOK)
    params = jnp.concatenate([starts, clens, jnp.zeros((16,), jnp.int32)])
    out = _pad_rows(params, flat_ids)
    padded = out.reshape(B, W)[:, :PADLEN]
    type_ids = jnp.zeros_like(padded)
    return padded, type_ids


# speedup vs baseline: 10.0169x; 1.0101x over previous
import functools

import jax
import jax.numpy as jnp
from jax import lax
from jax.experimental import pallas as pl
from jax.experimental.pallas import tpu as pltpu
from jax.experimental.pallas import tpu_sc as plsc

B = 16
TOTAL = 32768
CLS_ID = 101
SEP_ID = 102
PADLEN = 4098
MAXTOK = PADLEN - 2

H0 = 2056
H1 = PADLEN - H0
HALF = 2064
NVEC = HALF // 16
STAGE = HALF + 8
PAD = 8
BUF = STAGE + PAD + 16

_mesh = plsc.VectorSubcoreMesh(core_axis_name="c", subcore_axis_name="s")


@functools.partial(
    pl.kernel,
    out_type=(jax.ShapeDtypeStruct((B, PADLEN), jnp.int32),
              jax.ShapeDtypeStruct((B, PADLEN), jnp.int32)),
    mesh=_mesh,
    compiler_params=pltpu.CompilerParams(use_tc_tiling_on_sc=False),
    scratch_types=[
        pltpu.VMEM((32,), jnp.int32),
        pltpu.VMEM((BUF,), jnp.int32),
        pltpu.VMEM((HALF,), jnp.int32),
        pltpu.VMEM((HALF,), jnp.int32),
    ],
)
def _pad_rows(cu_hbm, flat_hbm, out_hbm, tid_hbm, cu_v, stage_v, row_v, zero_v):
    h = lax.axis_index("c")
    r = lax.axis_index("s")

    pltpu.sync_copy(cu_hbm, cu_v.at[pl.ds(0, B + 1)])
    start = cu_v[pl.ds(r, 16)][0]
    nxt = cu_v[pl.ds(r + 1, 16)][0]
    clen = jnp.minimum(nxt - start, MAXTOK)

    c0 = h * H0
    src_lo = start + c0 - 1
    abase = jnp.clip((jnp.maximum(src_lo, 0) // 8) * 8, 0, TOTAL - STAGE)
    abase = pl.multiple_of(abase, 8)
    pltpu.sync_copy(flat_hbm.at[pl.ds(abase, STAGE)], stage_v.at[pl.ds(PAD, STAGE)])

    shift = src_lo - abase + PAD
    sep_col = clen + 1
    lane = lax.iota(jnp.int32, 16)
    zero16 = jnp.zeros((16,), jnp.int32)

    def body(j, carry):
        l = j * 16 + lane
        col = c0 + l
        base = jnp.clip(shift + j * 16, 0, BUF - 16)
        tok = stage_v[pl.ds(base, 16)]
        val = jnp.where(col == 0, jnp.int32(CLS_ID),
              jnp.where(col == sep_col, jnp.int32(SEP_ID),
              jnp.where(col <= clen, tok, jnp.int32(0))))
        row_v[pl.ds(j * 16, 16)] = val
        zero_v[pl.ds(j * 16, 16)] = zero16
        return carry

    lax.fori_loop(0, NVEC, body, 0)

    @pl.when(h == 0)
    def _():
        pltpu.sync_copy(row_v.at[pl.ds(0, H0)], out_hbm.at[r, pl.ds(0, H0)])
        pltpu.sync_copy(zero_v.at[pl.ds(0, H0)], tid_hbm.at[r, pl.ds(0, H0)])

    @pl.when(h == 1)
    def _():
        pltpu.sync_copy(row_v.at[pl.ds(0, H1)], out_hbm.at[r, pl.ds(H0, H1)])
        pltpu.sync_copy(zero_v.at[pl.ds(0, H1)], tid_hbm.at[r, pl.ds(H0, H1)])


def kernel(flat_ids, cu_seqlens):
    return _pad_rows(cu_seqlens, flat_ids)
